# Initial kernel scaffold; baseline (speedup 1.0000x reference)
import jax
import jax.numpy as jnp
from jax.experimental import pallas as pl


def kernel(x, mask, W):
    raise NotImplementedError("write your pallas kernel here")


# speedup vs baseline: 2.9070x; 2.9070x over previous
import jax
import jax.numpy as jnp
from jax.experimental import pallas as pl

_TOPK = 8
_BIG_I32 = 2**31 - 1


def _wta_block(x_ref, m_ref, w_ref, o_ref):
    xb = x_ref[...]
    mb = m_ref[...] != 0
    wrow = w_ref[...]

    wmax = jnp.max(wrow, axis=1, keepdims=True)
    we = jnp.exp(wrow - wmax)
    softw = we / jnp.sum(we, axis=1, keepdims=True)

    masked = jnp.where(mb, xb, -jnp.inf)
    lanes = jax.lax.broadcasted_iota(jnp.int32, xb.shape, 1)
    for _ in range(_TOPK):
        cur = jnp.max(masked, axis=1, keepdims=True)
        idx = jnp.min(
            jnp.where(masked == cur, lanes, _BIG_I32), axis=1, keepdims=True
        )
        masked = jnp.where(lanes == idx, -jnp.inf, masked)
    sel = mb & (masked == -jnp.inf)
    sum_vw = jnp.sum(jnp.where(sel, xb * softw, 0.0), axis=1, keepdims=True)
    sum_v = jnp.sum(jnp.where(sel, xb, 0.0), axis=1, keepdims=True)
    o_ref[...] = sum_vw + sum_v * (1.0 / _TOPK)


def kernel(x, mask, W):
    B, N = x.shape
    R = 32
    wrow = W.reshape(1, N)
    mask_i8 = mask.astype(jnp.int8)
    out = pl.pallas_call(
        _wta_block,
        grid=(B // R,),
        in_specs=[
            pl.BlockSpec((R, N), lambda i: (i, 0)),
            pl.BlockSpec((R, N), lambda i: (i, 0)),
            pl.BlockSpec((1, N), lambda i: (0, 0)),
        ],
        out_specs=pl.BlockSpec((R, 1), lambda i: (i, 0)),
        out_shape=jax.ShapeDtypeStruct((B, 1), jnp.float32),
    )(x, mask_i8, wrow)
    return out
